# Initial kernel scaffold; baseline (speedup 1.0000x reference)
import jax
import jax.numpy as jnp
from jax.experimental import pallas as pl


def kernel(class_logits, box_regression, theta_preds, proposals):
    raise NotImplementedError("write your pallas kernel here")


# speedup vs baseline: 12.2674x; 12.2674x over previous
import jax
import jax.numpy as jnp
import jax.lax as lax
import numpy as np
from jax.experimental import pallas as pl
from jax.experimental.pallas import tpu as pltpu

_N = 1000
_C = 91
_NT = 9
_IMG_H = 800.0
_IMG_W = 1216.0
_ST = 0.05
_NMS = 0.5
_DETS = 100
_K = 2048
_CLIP = float(np.log(1000.0 / 16.0))
_CP = 96
_NP = 1024
_NEG = -1e30
_BIGI = np.int32(1 << 20)


def _body(lg_ref, dx_ref, dy_ref, dw_ref, dh_ref, pr_ref, th_ref, out_ref,
          sm_s, x1_s, y1_s, x2_s, y2_s, av_s, u_s):
    ki = lax.broadcasted_iota(jnp.int32, (_CP, _NP), 0)
    ri = lax.broadcasted_iota(jnp.int32, (_CP, _NP), 1)
    fid = ri * 90 + (ki - 1)

    L = lg_ref[...]
    mx = jnp.max(L, axis=0, keepdims=True)
    e = jnp.exp(L - mx)
    den = jnp.sum(e, axis=0, keepdims=True)
    sf = e / den

    px1 = pr_ref[0:1, :]
    py1 = pr_ref[1:2, :]
    px2 = pr_ref[2:3, :]
    py2 = pr_ref[3:4, :]
    w = px2 - px1
    h = py2 - py1
    cx = px1 + 0.5 * w
    cy = py1 + 0.5 * h
    ax = dx_ref[...] / 10.0
    ay = dy_ref[...] / 10.0
    aw = jnp.minimum(dw_ref[...] / 5.0, _CLIP)
    ah = jnp.minimum(dh_ref[...] / 5.0, _CLIP)
    pcx = ax * w + cx
    pcy = ay * h + cy
    pw = jnp.exp(aw) * w
    ph = jnp.exp(ah) * h
    bx1 = jnp.clip(pcx - 0.5 * pw, 0.0, _IMG_W)
    by1 = jnp.clip(pcy - 0.5 * ph, 0.0, _IMG_H)
    bx2 = jnp.clip(pcx + 0.5 * pw, 0.0, _IMG_W)
    by2 = jnp.clip(pcy + 0.5 * ph, 0.0, _IMG_H)

    ws = bx2 - bx1
    hs = by2 - by1
    act = (ki >= 1) & (ki <= _C - 1) & (ri < _N)
    valid = act & (ws >= 1e-2) & (hs >= 1e-2) & (sf > _ST)
    sm = jnp.where(valid, sf, -1.0)

    x1_s[...] = bx1
    y1_s[...] = by1
    x2_s[...] = bx2
    y2_s[...] = by2
    sm_s[...] = sm
    u = jnp.where(sm > 0.0, lax.bitcast_convert_type(sm, jnp.int32), 0)
    u_s[...] = u

    def v_step(i, T):
        b = 30 - i
        T2 = T | (jnp.int32(1) << b)
        c = jnp.sum((u_s[...] >= T2).astype(jnp.int32))
        return jnp.where(c >= _K, T2, T)

    V = lax.fori_loop(0, 31, v_step, jnp.int32(0))
    cnt_gt = jnp.sum((u_s[...] > V).astype(jnp.int32))
    need = _K - cnt_gt

    def m_step(i, M):
        b = 16 - i
        M2 = M | (jnp.int32(1) << b)
        c = jnp.sum(((u_s[...] == V) & (fid < M2)).astype(jnp.int32))
        return jnp.where(c < need, M2, M)

    M = lax.fori_loop(0, 17, m_step, jnp.int32(0))

    selm = (u > V) | ((V > 0) & (u == V) & (fid <= M))
    av_s[...] = jnp.where(selm & (sm > 0.0), sm, _NEG)

    lane = lax.broadcasted_iota(jnp.int32, (1, _NP), 1)

    def nms_step(i, fid0):
        av = av_s[...]
        mxv = jnp.max(av)
        empty = mxv <= 0.0
        cand = jnp.where(av == mxv, fid, _BIGI)
        j = jnp.where(empty, fid0, jnp.min(cand))
        kj = j % 90 + 1
        rj = j // 90
        onr = lane == rj
        x1r = x1_s[pl.ds(kj, 1), :]
        y1r = y1_s[pl.ds(kj, 1), :]
        x2r = x2_s[pl.ds(kj, 1), :]
        y2r = y2_s[pl.ds(kj, 1), :]
        smr = sm_s[pl.ds(kj, 1), :]
        sx1 = jnp.sum(jnp.where(onr, x1r, 0.0))
        sy1 = jnp.sum(jnp.where(onr, y1r, 0.0))
        sx2 = jnp.sum(jnp.where(onr, x2r, 0.0))
        sy2 = jnp.sum(jnp.where(onr, y2r, 0.0))
        ssc = jnp.sum(jnp.where(onr, smr, 0.0))
        out_ref[pl.ds(i, 1), pl.ds(0, 1)] = sx1.reshape(1, 1)
        out_ref[pl.ds(i, 1), pl.ds(1, 1)] = sy1.reshape(1, 1)
        out_ref[pl.ds(i, 1), pl.ds(2, 1)] = sx2.reshape(1, 1)
        out_ref[pl.ds(i, 1), pl.ds(3, 1)] = sy2.reshape(1, 1)
        out_ref[pl.ds(i, 1), pl.ds(4, 1)] = ssc.reshape(1, 1)
        thr = th_ref[pl.ds(rj, 1), :]
        out_ref[pl.ds(i, 1), pl.ds(5, 8)] = thr[:, 1:9]
        arear = (x2r - x1r) * (y2r - y1r)
        areaj = (sx2 - sx1) * (sy2 - sy1)
        ltx = jnp.maximum(x1r, sx1)
        lty = jnp.maximum(y1r, sy1)
        rbx = jnp.minimum(x2r, sx2)
        rby = jnp.minimum(y2r, sy2)
        iw = jnp.maximum(rbx - ltx, 0.0)
        ih = jnp.maximum(rby - lty, 0.0)
        inter = iw * ih
        iou = inter / (arear + areaj - inter + 1e-9)
        avr = av_s[pl.ds(kj, 1), :]
        av_s[pl.ds(kj, 1), :] = jnp.where((iou > _NMS) | onr, _NEG, avr)
        return jnp.where(i == 0, j, fid0)

    lax.fori_loop(0, _DETS, nms_step, jnp.int32(0))

    dat = out_ref[...]
    ok = (dat[:, 4:5] > 0.0).astype(jnp.float32)
    out_ref[...] = dat * ok


def kernel(class_logits, box_regression, theta_preds, proposals):
    lgT = jnp.pad(class_logits.T, ((0, _CP - _C), (0, _NP - _N)),
                  constant_values=-1e30)
    reg = box_regression.reshape(_N, _C, 4)

    def t(a):
        return jnp.pad(a.T, ((0, _CP - _C), (0, _NP - _N)))

    dxT = t(reg[:, :, 0])
    dyT = t(reg[:, :, 1])
    dwT = t(reg[:, :, 2])
    dhT = t(reg[:, :, 3])
    prT = jnp.pad(proposals.T, ((0, 4), (0, _NP - _N)))
    thP = jnp.pad(theta_preds, ((0, 0), (0, 16 - _NT)))

    return pl.pallas_call(
        _body,
        out_shape=jax.ShapeDtypeStruct((_DETS, 13), jnp.float32),
        scratch_shapes=[pltpu.VMEM((_CP, _NP), jnp.float32)] * 6
                       + [pltpu.VMEM((_CP, _NP), jnp.int32)],
    )(lgT, dxT, dyT, dwT, dhT, prT, thP)
